# initial kernel scaffold (unmeasured)
import jax
import jax.numpy as jnp
from jax import lax
from jax.experimental import pallas as pl
from jax.experimental.pallas import tpu as pltpu


def kernel(
    x,
):
    def body(*refs):
        pass

    out_shape = jax.ShapeDtypeStruct(..., jnp.float32)
    return pl.pallas_call(body, out_shape=out_shape)(...)


# baseline (device time: 30368 ns/iter reference)
import jax
import jax.numpy as jnp
from jax import lax
from jax.experimental import pallas as pl
from jax.experimental.pallas import tpu as pltpu


def kernel(x):
    m, n = x.shape
    n_half = n // 2

    def body(x_ref, out_ref, send_sem, recv_sem):
        mx = lax.axis_index("x")
        my = lax.axis_index("y")
        other_x = 1 - mx

        barrier_sem = pltpu.get_barrier_semaphore()
        pl.semaphore_signal(
            barrier_sem, inc=1,
            device_id=(other_x, my), device_id_type=pl.DeviceIdType.MESH,
        )
        pl.semaphore_wait(barrier_sem, 1)

        out_ref[pl.ds(mx * m, m), :] = x_ref[:, pl.ds(mx * n_half, n_half)]

        rdma = pltpu.make_async_remote_copy(
            src_ref=x_ref.at[:, pl.ds(other_x * n_half, n_half)],
            dst_ref=out_ref.at[pl.ds(mx * m, m), :],
            send_sem=send_sem,
            recv_sem=recv_sem,
            device_id=(other_x, my),
            device_id_type=pl.DeviceIdType.MESH,
        )
        rdma.start()
        rdma.wait()

    return pl.pallas_call(
        body,
        out_shape=jax.ShapeDtypeStruct((2 * m, n_half), x.dtype),
        in_specs=[pl.BlockSpec(memory_space=pltpu.VMEM)],
        out_specs=pl.BlockSpec(memory_space=pltpu.VMEM),
        scratch_shapes=[
            pltpu.SemaphoreType.DMA,
            pltpu.SemaphoreType.DMA,
        ],
        compiler_params=pltpu.CompilerParams(collective_id=0),
    )(x)


# device time: 24309 ns/iter; 1.2492x vs baseline; 1.2492x over previous
import jax
import jax.numpy as jnp
from jax import lax
from jax.experimental import pallas as pl
from jax.experimental.pallas import tpu as pltpu

NCHUNK = 8


def kernel(x):
    m, n = x.shape
    n_half = n // 2
    ch = m // NCHUNK
    nk = NCHUNK // 2

    def body(x_ref, out_ref, xsend_sems, xrecv_sems, ysend_sems, yrecv_sems):
        mx = lax.axis_index("x")
        my = lax.axis_index("y")
        ox = 1 - mx
        oy = 1 - my

        barrier_sem = pltpu.get_barrier_semaphore()
        pl.semaphore_signal(
            barrier_sem, inc=1,
            device_id=(ox, my), device_id_type=pl.DeviceIdType.MESH,
        )
        pl.semaphore_signal(
            barrier_sem, inc=1,
            device_id=(mx, oy), device_id_type=pl.DeviceIdType.MESH,
        )
        pl.semaphore_wait(barrier_sem, 2)

        row0 = ox * m

        xsends = []
        for k in range(nk):
            c = 2 * k + my
            rdma = pltpu.make_async_remote_copy(
                src_ref=x_ref.at[pl.ds(c * ch, ch), pl.ds(ox * n_half, n_half)],
                dst_ref=out_ref.at[pl.ds(mx * m + c * ch, ch), :],
                send_sem=xsend_sems.at[k],
                recv_sem=xrecv_sems.at[k],
                device_id=(ox, my),
                device_id_type=pl.DeviceIdType.MESH,
            )
            rdma.start()
            xsends.append(rdma)

        out_ref[pl.ds(mx * m, m), :] = x_ref[:, pl.ds(mx * n_half, n_half)]

        fwds = []
        for k in range(nk):
            c = 2 * k + my
            xsends[k].wait_recv()
            sl = out_ref.at[pl.ds(row0 + c * ch, ch), :]
            fwd = pltpu.make_async_remote_copy(
                src_ref=sl,
                dst_ref=sl,
                send_sem=ysend_sems.at[k],
                recv_sem=yrecv_sems.at[k],
                device_id=(mx, oy),
                device_id_type=pl.DeviceIdType.MESH,
            )
            fwd.start()
            fwds.append(fwd)

        for k in range(nk):
            fwds[k].wait_recv()
        for k in range(nk):
            xsends[k].wait_send()
            fwds[k].wait_send()

    return pl.pallas_call(
        body,
        out_shape=jax.ShapeDtypeStruct((2 * m, n_half), x.dtype),
        in_specs=[pl.BlockSpec(memory_space=pltpu.VMEM)],
        out_specs=pl.BlockSpec(memory_space=pltpu.VMEM),
        scratch_shapes=[
            pltpu.SemaphoreType.DMA((nk,)),
            pltpu.SemaphoreType.DMA((nk,)),
            pltpu.SemaphoreType.DMA((nk,)),
            pltpu.SemaphoreType.DMA((nk,)),
        ],
        compiler_params=pltpu.CompilerParams(collective_id=0),
    )(x)


# device time: 23109 ns/iter; 1.3141x vs baseline; 1.0519x over previous
import jax
import jax.numpy as jnp
from jax import lax
from jax.experimental import pallas as pl
from jax.experimental.pallas import tpu as pltpu

NCHUNK = 16


def kernel(x):
    m, n = x.shape
    n_half = n // 2
    ch = m // NCHUNK
    nk = NCHUNK // 2

    def body(x_ref, out_ref, xsend_sems, xrecv_sems, ysend_sems, yrecv_sems,
             copy_sem):
        mx = lax.axis_index("x")
        my = lax.axis_index("y")
        ox = 1 - mx
        oy = 1 - my

        barrier_sem = pltpu.get_barrier_semaphore()
        pl.semaphore_signal(
            barrier_sem, inc=1,
            device_id=(ox, my), device_id_type=pl.DeviceIdType.MESH,
        )
        pl.semaphore_signal(
            barrier_sem, inc=1,
            device_id=(mx, oy), device_id_type=pl.DeviceIdType.MESH,
        )
        pl.semaphore_wait(barrier_sem, 2)

        row0 = ox * m

        xsends = []
        for k in range(nk):
            c = 2 * k + my
            rdma = pltpu.make_async_remote_copy(
                src_ref=x_ref.at[pl.ds(c * ch, ch), pl.ds(ox * n_half, n_half)],
                dst_ref=out_ref.at[pl.ds(mx * m + c * ch, ch), :],
                send_sem=xsend_sems.at[k],
                recv_sem=xrecv_sems.at[k],
                device_id=(ox, my),
                device_id_type=pl.DeviceIdType.MESH,
            )
            rdma.start()
            xsends.append(rdma)

        local_copy = pltpu.make_async_copy(
            x_ref.at[:, pl.ds(mx * n_half, n_half)],
            out_ref.at[pl.ds(mx * m, m), :],
            copy_sem,
        )
        local_copy.start()

        fwds = []
        for k in range(nk):
            c = 2 * k + my
            xsends[k].wait_recv()
            sl = out_ref.at[pl.ds(row0 + c * ch, ch), :]
            fwd = pltpu.make_async_remote_copy(
                src_ref=sl,
                dst_ref=sl,
                send_sem=ysend_sems.at[k],
                recv_sem=yrecv_sems.at[k],
                device_id=(mx, oy),
                device_id_type=pl.DeviceIdType.MESH,
            )
            fwd.start()
            fwds.append(fwd)

        for k in range(nk):
            fwds[k].wait_recv()
        for k in range(nk):
            xsends[k].wait_send()
            fwds[k].wait_send()
        local_copy.wait()

    return pl.pallas_call(
        body,
        out_shape=jax.ShapeDtypeStruct((2 * m, n_half), x.dtype),
        in_specs=[pl.BlockSpec(memory_space=pltpu.VMEM)],
        out_specs=pl.BlockSpec(memory_space=pltpu.VMEM),
        scratch_shapes=[
            pltpu.SemaphoreType.DMA((nk,)),
            pltpu.SemaphoreType.DMA((nk,)),
            pltpu.SemaphoreType.DMA((nk,)),
            pltpu.SemaphoreType.DMA((nk,)),
            pltpu.SemaphoreType.DMA,
        ],
        compiler_params=pltpu.CompilerParams(collective_id=0),
    )(x)
